# Initial kernel scaffold; baseline (speedup 1.0000x reference)
import jax
import jax.numpy as jnp
from jax.experimental import pallas as pl


def kernel(positions, positional_embeddings):
    raise NotImplementedError("write your pallas kernel here")


# speedup vs baseline: 2.1880x; 2.1880x over previous
import functools

import jax
import jax.numpy as jnp
from jax import lax
from jax.experimental import pallas as pl
from jax.experimental.pallas import tpu as pltpu
from jax.experimental.pallas import tpu_sc as plsc

N_STATE = 1024

_NC = 2
_NS = 16
_NW = _NC * _NS

_B = 4 * 8192
_BPW = _B // _NW
_CHUNK = 64
_NCHUNK = _BPW // _CHUNK


def _make_gather():
    mesh = plsc.VectorSubcoreMesh(core_axis_name="c", subcore_axis_name="s")

    @functools.partial(
        pl.kernel,
        mesh=mesh,
        out_type=jax.ShapeDtypeStruct((_B, N_STATE), jnp.float32),
        scratch_types=[
            pltpu.VMEM((_BPW,), jnp.int32),
            pltpu.VMEM((_CHUNK, N_STATE), jnp.float32),
            pltpu.SemaphoreType.DMA,
        ],
    )
    def gather_kernel(idx_hbm, table_hbm, out_hbm, idx_v, rows_v, sem):
        wid = lax.axis_index("s") * _NC + lax.axis_index("c")
        base = wid * _BPW
        pltpu.sync_copy(idx_hbm.at[pl.ds(base, _BPW)], idx_v)

        def body(i, carry):
            off = i * _CHUNK
            pltpu.async_copy(
                table_hbm.at[idx_v.at[pl.ds(off, _CHUNK)]], rows_v, sem
            ).wait()
            pltpu.sync_copy(rows_v, out_hbm.at[pl.ds(base + off, _CHUNK)])
            return carry

        lax.fori_loop(0, _NCHUNK, body, 0)

    return gather_kernel


_gather = _make_gather()


@jax.jit
def kernel(positions, positional_embeddings):
    idx = positions.reshape(-1).astype(jnp.int32)
    out = _gather(idx, positional_embeddings)
    return out.reshape(positions.shape + (N_STATE,))


# speedup vs baseline: 2.3846x; 1.0899x over previous
import functools

import jax
import jax.numpy as jnp
from jax import lax
from jax.experimental import pallas as pl
from jax.experimental.pallas import tpu as pltpu
from jax.experimental.pallas import tpu_sc as plsc

N_STATE = 1024

_NC = 2
_NS = 16
_NW = _NC * _NS

_B = 4 * 8192
_BPW = _B // _NW
_CHUNK = 32
_NCHUNK = _BPW // _CHUNK


def _make_gather():
    mesh = plsc.VectorSubcoreMesh(core_axis_name="c", subcore_axis_name="s")

    @functools.partial(
        pl.kernel,
        mesh=mesh,
        out_type=jax.ShapeDtypeStruct((_B, N_STATE), jnp.float32),
        scratch_types=[
            pltpu.VMEM((_BPW,), jnp.int32),
            pltpu.VMEM((_CHUNK, N_STATE), jnp.float32),
            pltpu.VMEM((_CHUNK, N_STATE), jnp.float32),
            pltpu.SemaphoreType.DMA,
            pltpu.SemaphoreType.DMA,
        ],
    )
    def gather_kernel(idx_hbm, table_hbm, out_hbm, idx_v, buf0, buf1,
                      sem0, sem1):
        wid = lax.axis_index("s") * _NC + lax.axis_index("c")
        base = wid * _BPW
        pltpu.sync_copy(idx_hbm.at[pl.ds(base, _BPW)], idx_v)

        bufs = (buf0, buf1)
        sems = (sem0, sem1)

        def start_gather(i, buf, sem):
            pltpu.async_copy(
                table_hbm.at[idx_v.at[pl.ds(i * _CHUNK, _CHUNK)]], buf, sem
            )

        def wait_gather(buf, sem):
            pltpu.make_async_copy(
                table_hbm.at[pl.ds(0, _CHUNK)], buf, sem
            ).wait()

        def write_out(i, buf):
            pltpu.sync_copy(buf, out_hbm.at[pl.ds(base + i * _CHUNK, _CHUNK)])

        start_gather(0, buf0, sem0)

        def body(g, carry):
            for b in range(2):
                i = 2 * g + b
                start_gather(i + 1, bufs[1 - b], sems[1 - b])
                wait_gather(bufs[b], sems[b])
                write_out(i, bufs[b])
            return carry

        lax.fori_loop(0, (_NCHUNK - 2) // 2, body, 0)

        i = _NCHUNK - 2
        start_gather(i + 1, buf1, sem1)
        wait_gather(buf0, sem0)
        write_out(i, buf0)
        wait_gather(buf1, sem1)
        write_out(_NCHUNK - 1, buf1)

    return gather_kernel


_gather = _make_gather()


@jax.jit
def kernel(positions, positional_embeddings):
    idx = positions.reshape(-1).astype(jnp.int32)
    out = _gather(idx, positional_embeddings)
    return out.reshape(positions.shape + (N_STATE,))
